# Initial kernel scaffold; baseline (speedup 1.0000x reference)
import jax
import jax.numpy as jnp
from jax.experimental import pallas as pl


def kernel(fore_map, fore_rep_encoded, target_embed, align_attns):
    raise NotImplementedError("write your pallas kernel here")


# speedup vs baseline: 1.1769x; 1.1769x over previous
import functools

import jax
import jax.numpy as jnp
from jax import lax
from jax.experimental import pallas as pl
from jax.experimental.pallas import tpu as pltpu
from jax.experimental.pallas import tpu_sc as plsc

_B, _T, _S, _H, _D = 2, 2048, 2048, 12, 768
_K = 204
_KPAD = 224
_NPAIR = _B * _KPAD
_NTILES = 32
_PPT = _NPAIR // _NTILES
_LANES = 16
_NCHUNK = _S // _LANES


def _wtopk_body(te_ref, fr_ref, vals_ref, idxs_ref):
    te = te_ref[...]
    fr = fr_ref[...]
    dot = lax.dot_general(te, fr, (((2,), (1,)), ((0,), (0,))),
                          preferred_element_type=jnp.float32)
    n1 = jnp.sqrt(jnp.sum(te * te, axis=2))
    nf = jnp.sqrt(jnp.sum(fr * fr, axis=1, keepdims=True))
    w = dot / (jnp.maximum(n1, 1e-8) * jnp.maximum(nf, 1e-8))

    iota_t = lax.broadcasted_iota(jnp.int32, (_B, _T), 1)
    iota_k = lax.broadcasted_iota(jnp.int32, (_B, 256), 1)
    neg = jnp.float32(-jnp.inf)

    def body(j, carry):
        wc, vals, idxs = carry
        m = jnp.max(wc, axis=1, keepdims=True)
        cand = jnp.where(wc == m, iota_t, _T)
        sel = jnp.min(cand, axis=1, keepdims=True)
        vals = jnp.where(iota_k == j, m, vals)
        idxs = jnp.where(iota_k == j, sel, idxs)
        wc = jnp.where(iota_t == sel, neg, wc)
        return wc, vals, idxs

    _, vals, idxs = lax.fori_loop(
        0, _K, body,
        (w, jnp.zeros((_B, 256), jnp.float32), jnp.zeros((_B, 256), jnp.int32)))
    vals_ref[...] = vals
    idxs_ref[...] = idxs


def _maxred_body(p_ref, o_ref):
    o_ref[...] = jnp.max(p_ref[...], axis=0, keepdims=True)


_GATHER_DNUMS = lax.GatherDimensionNumbers(
    offset_dims=(), collapsed_slice_dims=(0,), start_index_map=(0,))


def _shuffle(x, perm):
    return lax.gather(x, perm.reshape(_LANES, 1), _GATHER_DNUMS,
                      slice_sizes=(1,),
                      mode=lax.GatherScatterMode.PROMISE_IN_BOUNDS)


def _bfly(x, op):
    for sh in (1, 2, 4, 8):
        perm = lax.iota(jnp.int32, _LANES) ^ sh
        x = op(x, _shuffle(x, perm))
    return x


def _sc_gather_body(tbl_hbm, ridx_hbm, w_hbm, out_hbm,
                    wv, idx_v, rows_v, y_v, part_v, sem):
    cid = lax.axis_index("c")
    sid = lax.axis_index("s")
    wid = sid * 2 + cid

    pltpu.sync_copy(w_hbm, wv)

    def zero_body(c, carry):
        part_v[pl.ds(c * _LANES, _LANES)] = jnp.zeros((_LANES,), jnp.float32)
        return carry

    lax.fori_loop(0, _B * _NCHUNK, zero_body, 0)

    def pair_body(i, carry):
        p = wid * _PPT + i
        pltpu.sync_copy(ridx_hbm.at[p], idx_v)
        pltpu.async_copy(tbl_hbm.at[idx_v], rows_v, sem).wait()
        wscale = wv[pl.ds(p, _LANES)][0] * jnp.float32(1.0 / _H)
        b = p // _KPAD

        def chunk1(c, mm):
            mnv, mxv = mm
            off = c * _LANES
            acc = rows_v[0, pl.ds(off, _LANES)]
            for h in range(1, _H):
                acc = acc + rows_v[h, pl.ds(off, _LANES)]
            y = jnp.maximum(acc * wscale, jnp.float32(0.0))
            y_v[pl.ds(off, _LANES)] = y
            return jnp.minimum(mnv, y), jnp.maximum(mxv, y)

        big = jnp.full((_LANES,), 3.4e38, jnp.float32)
        mnv, mxv = lax.fori_loop(0, _NCHUNK, chunk1, (big, -big))
        mn = _bfly(mnv, jnp.minimum)
        mx = _bfly(mxv, jnp.maximum)
        d = jnp.minimum(jnp.maximum(mx - mn, jnp.float32(1e-12)),
                        jnp.float32(1.0))
        rcp = jnp.float32(1.0) / d

        def chunk2(c, carry2):
            off = c * _LANES
            y = y_v[pl.ds(off, _LANES)]
            val = (y - mn) * rcp
            q = pl.ds(b * _S + off, _LANES)
            part_v[q] = jnp.maximum(part_v[q], val)
            return carry2

        lax.fori_loop(0, _NCHUNK, chunk2, 0)
        return carry

    lax.fori_loop(0, _PPT, pair_body, 0)
    pltpu.sync_copy(part_v, out_hbm.at[wid])


@functools.cache
def _make_sc_gather():
    return functools.partial(
        pl.kernel,
        mesh=plsc.VectorSubcoreMesh(core_axis_name="c", subcore_axis_name="s"),
        out_type=jax.ShapeDtypeStruct((_NTILES, _B * _S), jnp.float32),
        scratch_types=[
            pltpu.VMEM((_NPAIR + _LANES,), jnp.float32),
            pltpu.VMEM((_H,), jnp.int32),
            pltpu.VMEM((_H, _S), jnp.float32),
            pltpu.VMEM((_S,), jnp.float32),
            pltpu.VMEM((_B * _S,), jnp.float32),
            pltpu.SemaphoreType.DMA,
        ],
    )(_sc_gather_body)


def kernel(fore_map, fore_rep_encoded, target_embed, align_attns):
    vals_p, idxs_p = pl.pallas_call(
        _wtopk_body,
        out_shape=[jax.ShapeDtypeStruct((_B, 256), jnp.float32),
                   jax.ShapeDtypeStruct((_B, 256), jnp.int32)],
    )(target_embed, fore_rep_encoded)
    idxs = idxs_p[:, :_K]
    vals = vals_p[:, :_K]

    idx_pad = jnp.pad(idxs, ((0, 0), (0, _KPAD - _K)))
    w_pad = jnp.pad(vals, ((0, 0), (0, _KPAD - _K)))
    bh = (jnp.arange(_B, dtype=jnp.int32)[:, None, None] * _H
          + jnp.arange(_H, dtype=jnp.int32)[None, None, :]) * _T
    ridx = (bh + idx_pad[:, :, None]).reshape(_NPAIR, _H)
    wflat = jnp.pad(w_pad.reshape(_NPAIR), (0, _LANES))
    tbl = align_attns.reshape(_B * _H * _T, _S)

    partials = _make_sc_gather()(tbl, ridx, wflat)

    tot = pl.pallas_call(
        _maxred_body,
        out_shape=jax.ShapeDtypeStruct((1, _B * _S), jnp.float32),
    )(partials)
    total_attn = tot.reshape(_B, _S)
    fm = fore_map[:, 0, :]
    return (fm, total_attn, idxs)


# speedup vs baseline: 1.3351x; 1.1344x over previous
import functools

import jax
import jax.numpy as jnp
from jax import lax
from jax.experimental import pallas as pl
from jax.experimental.pallas import tpu as pltpu
from jax.experimental.pallas import tpu_sc as plsc

_B, _T, _S, _H, _D = 2, 2048, 2048, 12, 768
_K = 204
_KPAD = 208
_NPAIR = _B * _KPAD
_NTILES = 32
_PPT = _NPAIR // _NTILES
_LANES = 16
_NCHUNK = _S // _LANES


def _wtopk_body(te_ref, fr_ref, vals_ref, idxs_ref):
    te = te_ref[...]
    fr = fr_ref[...]
    dot = lax.dot_general(te, fr, (((2,), (1,)), ((0,), (0,))),
                          preferred_element_type=jnp.float32)
    n1 = jnp.sqrt(jnp.sum(te * te, axis=2))
    nf = jnp.sqrt(jnp.sum(fr * fr, axis=1, keepdims=True))
    w = dot / (jnp.maximum(n1, 1e-8) * jnp.maximum(nf, 1e-8))

    iota_t = lax.broadcasted_iota(jnp.int32, (_B, _T), 1)
    iota_k = lax.broadcasted_iota(jnp.int32, (_B, 256), 1)
    neg = jnp.float32(-jnp.inf)

    def body(j, carry):
        wc, vals, idxs = carry
        m = jnp.max(wc, axis=1, keepdims=True)
        cand = jnp.where(wc == m, iota_t, _T)
        sel = jnp.min(cand, axis=1, keepdims=True)
        vals = jnp.where(iota_k == j, m, vals)
        idxs = jnp.where(iota_k == j, sel, idxs)
        wc = jnp.where(iota_t == sel, neg, wc)
        return wc, vals, idxs

    _, vals, idxs = lax.fori_loop(
        0, _K, body,
        (w, jnp.zeros((_B, 256), jnp.float32), jnp.zeros((_B, 256), jnp.int32)))
    vals_ref[...] = vals
    idxs_ref[...] = idxs


def _maxred_body(p_ref, o_ref):
    o_ref[...] = jnp.max(p_ref[...], axis=0, keepdims=True)


_GATHER_DNUMS = lax.GatherDimensionNumbers(
    offset_dims=(), collapsed_slice_dims=(0,), start_index_map=(0,))


def _shuffle(x, perm):
    return lax.gather(x, perm.reshape(_LANES, 1), _GATHER_DNUMS,
                      slice_sizes=(1,),
                      mode=lax.GatherScatterMode.PROMISE_IN_BOUNDS)


def _bfly(x, op):
    for sh in (1, 2, 4, 8):
        perm = lax.iota(jnp.int32, _LANES) ^ sh
        x = op(x, _shuffle(x, perm))
    return x


def _sc_gather_body(tbl_hbm, ridx_hbm, w_hbm, out_hbm,
                    wv, idx0, idx1, rows0, rows1, y_v, part_v, sem0, sem1):
    cid = lax.axis_index("c")
    sid = lax.axis_index("s")
    wid = sid * 2 + cid
    base = wid * _PPT

    pltpu.sync_copy(w_hbm, wv)

    def zero_body(c, carry):
        part_v[pl.ds(c * _LANES, _LANES)] = jnp.zeros((_LANES,), jnp.float32)
        return carry

    lax.fori_loop(0, _B * _NCHUNK, zero_body, 0)

    def issue(i, idxb, rows, sem):
        pltpu.sync_copy(ridx_hbm.at[pl.ds((base + i) * _LANES, _H)], idxb)
        return pltpu.async_copy(tbl_hbm.at[idxb], rows, sem)

    def compute(i, rows):
        p = base + i
        wscale = wv[pl.ds(p, _LANES)][0] * jnp.float32(1.0 / _H)
        b = p // _KPAD

        def chunk1(c, mm):
            mnv, mxv = mm
            off = c * _LANES
            acc = rows[0, pl.ds(off, _LANES)]
            for h in range(1, _H):
                acc = acc + rows[h, pl.ds(off, _LANES)]
            y = jnp.maximum(acc * wscale, jnp.float32(0.0))
            y_v[pl.ds(off, _LANES)] = y
            return jnp.minimum(mnv, y), jnp.maximum(mxv, y)

        big = jnp.full((_LANES,), 3.4e38, jnp.float32)
        mnv, mxv = lax.fori_loop(0, _NCHUNK, chunk1, (big, -big),
                                 unroll=4)
        mn = _bfly(mnv, jnp.minimum)
        mx = _bfly(mxv, jnp.maximum)
        d = jnp.minimum(jnp.maximum(mx - mn, jnp.float32(1e-12)),
                        jnp.float32(1.0))
        rcp = jnp.float32(1.0) / d

        def chunk2(c, carry2):
            off = c * _LANES
            y = y_v[pl.ds(off, _LANES)]
            val = (y - mn) * rcp
            q = pl.ds(b * _S + off, _LANES)
            part_v[q] = jnp.maximum(part_v[q], val)
            return carry2

        lax.fori_loop(0, _NCHUNK, chunk2, 0, unroll=4)

    idxb = (idx0, idx1)
    rows = (rows0, rows1)
    sems = (sem0, sem1)
    handles = [None, None]
    handles[0] = issue(0, idx0, rows0, sem0)
    for i in range(_PPT):
        cur = i % 2
        nxt = (i + 1) % 2
        if i + 1 < _PPT:
            handles[nxt] = issue(i + 1, idxb[nxt], rows[nxt], sems[nxt])
        handles[cur].wait()
        compute(i, rows[cur])

    pltpu.sync_copy(part_v, out_hbm.at[wid])


@functools.cache
def _make_sc_gather():
    return functools.partial(
        pl.kernel,
        mesh=plsc.VectorSubcoreMesh(core_axis_name="c", subcore_axis_name="s"),
        out_type=jax.ShapeDtypeStruct((_NTILES, _B * _S), jnp.float32),
        scratch_types=[
            pltpu.VMEM((_NPAIR + _LANES,), jnp.float32),
            pltpu.VMEM((_H,), jnp.int32),
            pltpu.VMEM((_H,), jnp.int32),
            pltpu.VMEM((_H, _S), jnp.float32),
            pltpu.VMEM((_H, _S), jnp.float32),
            pltpu.VMEM((_S,), jnp.float32),
            pltpu.VMEM((_B * _S,), jnp.float32),
            pltpu.SemaphoreType.DMA,
            pltpu.SemaphoreType.DMA,
        ],
    )(_sc_gather_body)


def kernel(fore_map, fore_rep_encoded, target_embed, align_attns):
    vals_p, idxs_p = pl.pallas_call(
        _wtopk_body,
        out_shape=[jax.ShapeDtypeStruct((_B, 256), jnp.float32),
                   jax.ShapeDtypeStruct((_B, 256), jnp.int32)],
    )(target_embed, fore_rep_encoded)
    idxs = idxs_p[:, :_K]
    vals = vals_p[:, :_K]

    idx_pad = jnp.pad(idxs, ((0, 0), (0, _KPAD - _K)))
    w_pad = jnp.pad(vals, ((0, 0), (0, _KPAD - _K)))
    bh = (jnp.arange(_B, dtype=jnp.int32)[:, None, None] * _H
          + jnp.arange(_H, dtype=jnp.int32)[None, None, :]) * _T
    ridx = (bh + idx_pad[:, :, None]).reshape(_NPAIR, _H)
    ridx = jnp.pad(ridx, ((0, 0), (0, _LANES - _H))).reshape(_NPAIR * _LANES)
    wflat = jnp.pad(w_pad.reshape(_NPAIR), (0, _LANES))
    tbl = align_attns.reshape(_B * _H * _T, _S)

    partials = _make_sc_gather()(tbl, ridx, wflat)

    tot = pl.pallas_call(
        _maxred_body,
        out_shape=jax.ShapeDtypeStruct((1, _B * _S), jnp.float32),
    )(partials)
    total_attn = tot.reshape(_B, _S)
    fm = fore_map[:, 0, :]
    return (fm, total_attn, idxs)


# speedup vs baseline: 1.4033x; 1.0511x over previous
import functools

import jax
import jax.numpy as jnp
from jax import lax
from jax.experimental import pallas as pl
from jax.experimental.pallas import tpu as pltpu
from jax.experimental.pallas import tpu_sc as plsc

_B, _T, _S, _H, _D = 2, 2048, 2048, 12, 768
_K = 204
_KPAD = 208
_NPAIR = _B * _KPAD
_NTILES = 32
_PPT = _NPAIR // _NTILES
_LANES = 16
_NCHUNK = _S // _LANES


def _wtopk_body(te_ref, fr_ref, vals_ref, idxs_ref):
    te = te_ref[...]
    fr = fr_ref[...]
    dot = lax.dot_general(te, fr, (((2,), (1,)), ((0,), (0,))),
                          preferred_element_type=jnp.float32)
    n1 = jnp.sqrt(jnp.sum(te * te, axis=2))
    nf = jnp.sqrt(jnp.sum(fr * fr, axis=1, keepdims=True))
    w = dot / (jnp.maximum(n1, 1e-8) * jnp.maximum(nf, 1e-8))

    iota_t = lax.broadcasted_iota(jnp.int32, (_B, _T), 1)
    iota_k = lax.broadcasted_iota(jnp.int32, (_B, 256), 1)
    neg = jnp.float32(-jnp.inf)

    def body(j, carry):
        wc, vals, idxs = carry
        m = jnp.max(wc, axis=1, keepdims=True)
        cand = jnp.where(wc == m, iota_t, _T)
        sel = jnp.min(cand, axis=1, keepdims=True)
        vals = jnp.where(iota_k == j, m, vals)
        idxs = jnp.where(iota_k == j, sel, idxs)
        wc = jnp.where(iota_t == sel, neg, wc)
        return wc, vals, idxs

    _, vals, idxs = lax.fori_loop(
        0, _K, body,
        (w, jnp.zeros((_B, 256), jnp.float32), jnp.zeros((_B, 256), jnp.int32)))
    vals_ref[...] = vals
    idxs_ref[...] = idxs


def _maxred_body(p_ref, o_ref):
    o_ref[...] = jnp.max(p_ref[...], axis=0, keepdims=True)


_GATHER_DNUMS = lax.GatherDimensionNumbers(
    offset_dims=(), collapsed_slice_dims=(0,), start_index_map=(0,))


def _shuffle(x, perm):
    return lax.gather(x, perm.reshape(_LANES, 1), _GATHER_DNUMS,
                      slice_sizes=(1,),
                      mode=lax.GatherScatterMode.PROMISE_IN_BOUNDS)


def _bfly(x, op):
    for sh in (1, 2, 4, 8):
        perm = lax.iota(jnp.int32, _LANES) ^ sh
        x = op(x, _shuffle(x, perm))
    return x


def _sc_gather_body(tbl_hbm, ridx_hbm, w_hbm, out_hbm,
                    wv, idx0, idx1, rows0, rows1, y_v, part_v, sem0, sem1):
    cid = lax.axis_index("c")
    sid = lax.axis_index("s")
    wid = sid * 2 + cid
    base = wid * _PPT

    pltpu.sync_copy(w_hbm, wv)

    def zero_body(c, carry):
        part_v[pl.ds(c * _LANES, _LANES)] = jnp.zeros((_LANES,), jnp.float32)
        return carry

    lax.fori_loop(0, _B * _NCHUNK, zero_body, 0)

    def issue(i, idxb, rows, sem):
        pltpu.sync_copy(ridx_hbm.at[pl.ds((base + i) * _LANES, _H)], idxb)
        return pltpu.async_copy(tbl_hbm.at[idxb], rows, sem)

    def compute(i, rows):
        p = base + i
        wscale = wv[pl.ds(p, _LANES)][0] * jnp.float32(1.0 / _H)
        b = p // _KPAD

        def chunk1(c, mm):
            mnv, mxv = mm
            off = c * _LANES
            a0 = rows[0, pl.ds(off, _LANES)] + rows[4, pl.ds(off, _LANES)]
            a1 = rows[1, pl.ds(off, _LANES)] + rows[5, pl.ds(off, _LANES)]
            a2 = rows[2, pl.ds(off, _LANES)] + rows[6, pl.ds(off, _LANES)]
            a3 = rows[3, pl.ds(off, _LANES)] + rows[7, pl.ds(off, _LANES)]
            a0 = a0 + rows[8, pl.ds(off, _LANES)]
            a1 = a1 + rows[9, pl.ds(off, _LANES)]
            a2 = a2 + rows[10, pl.ds(off, _LANES)]
            a3 = a3 + rows[11, pl.ds(off, _LANES)]
            acc = (a0 + a1) + (a2 + a3)
            y = jnp.maximum(acc * wscale, jnp.float32(0.0))
            y_v[pl.ds(off, _LANES)] = y
            return jnp.minimum(mnv, y), jnp.maximum(mxv, y)

        big = jnp.full((_LANES,), 3.4e38, jnp.float32)
        mnv, mxv = lax.fori_loop(0, _NCHUNK, chunk1, (big, -big),
                                 unroll=4)
        mn = _bfly(mnv, jnp.minimum)
        mx = _bfly(mxv, jnp.maximum)
        d = jnp.minimum(jnp.maximum(mx - mn, jnp.float32(1e-12)),
                        jnp.float32(1.0))
        rcp = jnp.float32(1.0) / d

        def chunk2(c, carry2):
            off = c * _LANES
            y = y_v[pl.ds(off, _LANES)]
            val = (y - mn) * rcp
            q = pl.ds(b * _S + off, _LANES)
            part_v[q] = jnp.maximum(part_v[q], val)
            return carry2

        lax.fori_loop(0, _NCHUNK, chunk2, 0, unroll=4)

    rows = (rows0, rows1)
    sems = (sem0, sem1)
    idxb = (idx0, idx1)
    handles = [None, None]
    handles[0] = issue(0, idxb[0], rows0, sem0)
    for i in range(_PPT):
        cur = i % 2
        nxt = (i + 1) % 2
        if i + 1 < _PPT:
            handles[nxt] = issue(i + 1, idxb[nxt], rows[nxt], sems[nxt])
        handles[cur].wait()
        compute(i, rows[cur])

    pltpu.sync_copy(part_v, out_hbm.at[wid])


@functools.cache
def _make_sc_gather():
    return functools.partial(
        pl.kernel,
        mesh=plsc.VectorSubcoreMesh(core_axis_name="c", subcore_axis_name="s"),
        out_type=jax.ShapeDtypeStruct((_NTILES, _B * _S), jnp.float32),
        scratch_types=[
            pltpu.VMEM((_NPAIR + _LANES,), jnp.float32),
            pltpu.VMEM((_H,), jnp.int32),
            pltpu.VMEM((_H,), jnp.int32),
            pltpu.VMEM((_H, _S), jnp.float32),
            pltpu.VMEM((_H, _S), jnp.float32),
            pltpu.VMEM((_S,), jnp.float32),
            pltpu.VMEM((_B * _S,), jnp.float32),
            pltpu.SemaphoreType.DMA,
            pltpu.SemaphoreType.DMA,
        ],
    )(_sc_gather_body)


def kernel(fore_map, fore_rep_encoded, target_embed, align_attns):
    vals_p, idxs_p = pl.pallas_call(
        _wtopk_body,
        out_shape=[jax.ShapeDtypeStruct((_B, 256), jnp.float32),
                   jax.ShapeDtypeStruct((_B, 256), jnp.int32)],
    )(target_embed, fore_rep_encoded)
    idxs = idxs_p[:, :_K]
    vals = vals_p[:, :_K]

    idx_pad = jnp.pad(idxs, ((0, 0), (0, _KPAD - _K)))
    w_pad = jnp.pad(vals, ((0, 0), (0, _KPAD - _K)))
    bh = (jnp.arange(_B, dtype=jnp.int32)[:, None, None] * _H
          + jnp.arange(_H, dtype=jnp.int32)[None, None, :]) * _T
    ridx = (bh + idx_pad[:, :, None]).reshape(_NPAIR, _H)
    ridx = jnp.pad(ridx, ((0, 0), (0, _LANES - _H))).reshape(_NPAIR * _LANES)
    wflat = jnp.pad(w_pad.reshape(_NPAIR), (0, _LANES))
    tbl = align_attns.reshape(_B * _H * _T, _S)

    partials = _make_sc_gather()(tbl, ridx, wflat)

    tot = pl.pallas_call(
        _maxred_body,
        out_shape=jax.ShapeDtypeStruct((1, _B * _S), jnp.float32),
    )(partials)
    total_attn = tot.reshape(_B, _S)
    fm = fore_map[:, 0, :]
    return (fm, total_attn, idxs)
